# initial kernel scaffold (unmeasured)
import jax
import jax.numpy as jnp
from jax import lax
from jax.experimental import pallas as pl
from jax.experimental.pallas import tpu as pltpu

N_DEV = 32


def kernel(x, w_mat):
    m_per, k = x.shape
    _, n = w_mat.shape

    def body(x_ref, w_ref, out_ref, comm_ref, send_sems, recv_sems, credit_sem):
        my_pos = lax.axis_index("i")
        left = lax.rem(my_pos - 1 + N_DEV, N_DEV)
        right = lax.rem(my_pos + 1, N_DEV)

        barrier_sem = pltpu.get_barrier_semaphore()
        for nbr in (left, right):
            pl.semaphore_signal(
                barrier_sem, inc=1,
                device_id=(nbr,), device_id_type=pl.DeviceIdType.MESH,
            )
        pl.semaphore_wait(barrier_sem, 2)

        comm_ref[0, :, :] = x_ref[:, :]

        for h in range(N_DEV - 1):
            send_slot = h % 2
            recv_slot = (h + 1) % 2
            if h >= 2:
                pl.semaphore_wait(credit_sem, 1)
            rdma = pltpu.make_async_remote_copy(
                src_ref=comm_ref.at[send_slot],
                dst_ref=comm_ref.at[recv_slot],
                send_sem=send_sems.at[send_slot],
                recv_sem=recv_sems.at[recv_slot],
                device_id=(right,),
                device_id_type=pl.DeviceIdType.MESH,
            )
            rdma.start()
            if h == 0:
                own = jnp.dot(x_ref[:, :], w_ref[:, :],
                              preferred_element_type=jnp.float32)
                out_ref[pl.ds(my_pos * m_per, m_per), :] = own
            rdma.wait()

            origin = lax.rem(my_pos - h - 1 + N_DEV, N_DEV)
            blk = jnp.dot(comm_ref[recv_slot, :, :], w_ref[:, :],
                          preferred_element_type=jnp.float32)
            out_ref[pl.ds(origin * m_per, m_per), :] = blk
            pl.semaphore_signal(
                credit_sem, inc=1,
                device_id=(left,), device_id_type=pl.DeviceIdType.MESH,
            )

    return pl.pallas_call(
        body,
        out_shape=jax.ShapeDtypeStruct((N_DEV * m_per, n), jnp.float32),
        in_specs=[
            pl.BlockSpec(memory_space=pltpu.VMEM),
            pl.BlockSpec(memory_space=pltpu.VMEM),
        ],
        out_specs=pl.BlockSpec(memory_space=pltpu.VMEM),
        scratch_shapes=[
            pltpu.VMEM((2, m_per, k), jnp.float32),
            pltpu.SemaphoreType.DMA((2,)),
            pltpu.SemaphoreType.DMA((2,)),
            pltpu.SemaphoreType.REGULAR,
        ],
        compiler_params=pltpu.CompilerParams(collective_id=0),
    )(x, w_mat)


# baseline (device time: 776462 ns/iter reference)
import jax
import jax.numpy as jnp
from jax import lax
from jax.experimental import pallas as pl
from jax.experimental.pallas import tpu as pltpu

N_DEV = 32


def kernel(x, w_mat):
    m_per, k = x.shape
    _, n = w_mat.shape

    def body(x_ref, w_ref, out_ref, comm_ref, send_sems, recv_sems):
        my_pos = lax.axis_index("i")
        left = lax.rem(my_pos - 1 + N_DEV, N_DEV)
        right = lax.rem(my_pos + 1, N_DEV)

        barrier_sem = pltpu.get_barrier_semaphore()
        for nbr in (left, right):
            pl.semaphore_signal(
                barrier_sem, inc=1,
                device_id=(nbr,), device_id_type=pl.DeviceIdType.MESH,
            )
        pl.semaphore_wait(barrier_sem, 2)

        comm_ref[0, :, :] = x_ref[:, :]

        for h in range(N_DEV - 1):
            send_slot = h % 2
            recv_slot = (h + 1) % 2
            rdma = pltpu.make_async_remote_copy(
                src_ref=comm_ref.at[send_slot],
                dst_ref=comm_ref.at[recv_slot],
                send_sem=send_sems.at[send_slot],
                recv_sem=recv_sems.at[recv_slot],
                device_id=(right,),
                device_id_type=pl.DeviceIdType.MESH,
            )
            rdma.start()
            if h == 0:
                own = jnp.dot(x_ref[:, :], w_ref[:, :],
                              preferred_element_type=jnp.float32)
                out_ref[pl.ds(my_pos * m_per, m_per), :] = own
            rdma.wait()

            origin = lax.rem(my_pos - h - 1 + N_DEV, N_DEV)
            blk = jnp.dot(comm_ref[recv_slot, :, :], w_ref[:, :],
                          preferred_element_type=jnp.float32)
            out_ref[pl.ds(origin * m_per, m_per), :] = blk

    return pl.pallas_call(
        body,
        out_shape=jax.ShapeDtypeStruct((N_DEV * m_per, n), jnp.float32),
        in_specs=[
            pl.BlockSpec(memory_space=pltpu.VMEM),
            pl.BlockSpec(memory_space=pltpu.VMEM),
        ],
        out_specs=pl.BlockSpec(memory_space=pltpu.VMEM),
        scratch_shapes=[
            pltpu.VMEM((2, m_per, k), jnp.float32),
            pltpu.SemaphoreType.DMA((2,)),
            pltpu.SemaphoreType.DMA((2,)),
        ],
        compiler_params=pltpu.CompilerParams(collective_id=0),
    )(x, w_mat)


# device time: 416966 ns/iter; 1.8622x vs baseline; 1.8622x over previous
import numpy as np

import jax
import jax.numpy as jnp
from jax import lax
from jax.experimental import pallas as pl
from jax.experimental.pallas import tpu as pltpu

N_DEV = 32


def _ring_tables():
    yz_cycle = [
        (0, 0), (0, 1), (0, 2), (0, 3), (1, 3), (1, 2), (1, 1), (2, 1),
        (2, 2), (2, 3), (3, 3), (3, 2), (3, 1), (3, 0), (2, 0), (1, 0),
    ]
    q_of = {(0, 0): 0, (1, 0): 1, (1, 1): 2, (0, 1): 3,
            (0, 2): 4, (1, 2): 5, (1, 3): 6, (0, 3): 7}
    ring = []
    for i, (y, z) in enumerate(yz_cycle):
        for j in (0, 1):
            x = (i + j) % 2
            ring.append(z * 8 + q_of[(x, y)])
    assert sorted(ring) == list(range(N_DEV))
    inv = [0] * N_DEV
    for r, p in enumerate(ring):
        inv[p] = r
    return np.array(ring, np.int32), np.array(inv, np.int32)


_RING, _INV = _ring_tables()


def kernel(x, w_mat):
    m_per, k = x.shape
    _, n = w_mat.shape
    half = m_per // 2

    ring = jnp.asarray(_RING)
    inv = jnp.asarray(_INV)
    my_pos = lax.axis_index("i")
    my_r = inv[my_pos]
    hs = jnp.arange(N_DEV - 1, dtype=jnp.int32)
    nbrs = jnp.stack([
        ring[(my_r + 1) % N_DEV],
        ring[(my_r - 1) % N_DEV],
    ]).astype(jnp.int32)
    rows_r = (ring[(my_r - hs - 1) % N_DEV] * m_per).astype(jnp.int32)
    rows_l = (ring[(my_r + hs + 1) % N_DEV] * m_per + half).astype(jnp.int32)

    def body(nbr_ref, rows_r_ref, rows_l_ref, x_ref, w_ref, out_ref,
             comm_r, comm_l, send_r, recv_r, send_l, recv_l):
        pos = lax.axis_index("i")
        right = nbr_ref[0]
        left = nbr_ref[1]

        barrier_sem = pltpu.get_barrier_semaphore()
        for nbr in (left, right):
            pl.semaphore_signal(
                barrier_sem, inc=1,
                device_id=(nbr,), device_id_type=pl.DeviceIdType.MESH,
            )
        pl.semaphore_wait(barrier_sem, 2)

        comm_r[0, :, :] = x_ref[:half, :]
        comm_l[0, :, :] = x_ref[half:, :]

        def gemm_halves(h):
            slot = (h + 1) % 2
            blk_r = jnp.dot(comm_r[slot, :, :], w_ref[:, :],
                            preferred_element_type=jnp.float32)
            out_ref[pl.ds(rows_r_ref[h], half), :] = blk_r
            blk_l = jnp.dot(comm_l[slot, :, :], w_ref[:, :],
                            preferred_element_type=jnp.float32)
            out_ref[pl.ds(rows_l_ref[h], half), :] = blk_l

        for h in range(N_DEV - 1):
            send_slot = h % 2
            recv_slot = (h + 1) % 2
            rdma_r = pltpu.make_async_remote_copy(
                src_ref=comm_r.at[send_slot],
                dst_ref=comm_r.at[recv_slot],
                send_sem=send_r.at[send_slot],
                recv_sem=recv_r.at[recv_slot],
                device_id=(right,),
                device_id_type=pl.DeviceIdType.MESH,
            )
            rdma_l = pltpu.make_async_remote_copy(
                src_ref=comm_l.at[send_slot],
                dst_ref=comm_l.at[recv_slot],
                send_sem=send_l.at[send_slot],
                recv_sem=recv_l.at[recv_slot],
                device_id=(left,),
                device_id_type=pl.DeviceIdType.MESH,
            )
            rdma_r.start()
            rdma_l.start()
            if h == 0:
                own = jnp.dot(x_ref[:, :], w_ref[:, :],
                              preferred_element_type=jnp.float32)
                out_ref[pl.ds(pos * m_per, m_per), :] = own
            else:
                gemm_halves(h - 1)
            rdma_r.wait()
            rdma_l.wait()

        gemm_halves(N_DEV - 2)

    return pl.pallas_call(
        body,
        out_shape=jax.ShapeDtypeStruct((N_DEV * m_per, n), jnp.float32),
        in_specs=[
            pl.BlockSpec(memory_space=pltpu.SMEM),
            pl.BlockSpec(memory_space=pltpu.SMEM),
            pl.BlockSpec(memory_space=pltpu.SMEM),
            pl.BlockSpec(memory_space=pltpu.VMEM),
            pl.BlockSpec(memory_space=pltpu.VMEM),
        ],
        out_specs=pl.BlockSpec(memory_space=pltpu.VMEM),
        scratch_shapes=[
            pltpu.VMEM((2, half, k), jnp.float32),
            pltpu.VMEM((2, half, k), jnp.float32),
            pltpu.SemaphoreType.DMA((2,)),
            pltpu.SemaphoreType.DMA((2,)),
            pltpu.SemaphoreType.DMA((2,)),
            pltpu.SemaphoreType.DMA((2,)),
        ],
        compiler_params=pltpu.CompilerParams(collective_id=0),
    )(nbrs, rows_r, rows_l, x, w_mat)


# device time: 364772 ns/iter; 2.1286x vs baseline; 1.1431x over previous
import numpy as np

import jax
import jax.numpy as jnp
from jax import lax
from jax.experimental import pallas as pl
from jax.experimental.pallas import tpu as pltpu

N_DEV = 32
NSLOT = 3


def _ring_tables():
    yz_cycle = [
        (0, 0), (0, 1), (0, 2), (0, 3), (1, 3), (1, 2), (1, 1), (2, 1),
        (2, 2), (2, 3), (3, 3), (3, 2), (3, 1), (3, 0), (2, 0), (1, 0),
    ]
    q_of = {(0, 0): 0, (1, 0): 1, (1, 1): 2, (0, 1): 3,
            (0, 2): 4, (1, 2): 5, (1, 3): 6, (0, 3): 7}
    ring = []
    for i, (y, z) in enumerate(yz_cycle):
        for j in (0, 1):
            x = (i + j) % 2
            ring.append(z * 8 + q_of[(x, y)])
    assert sorted(ring) == list(range(N_DEV))
    inv = [0] * N_DEV
    for r, p in enumerate(ring):
        inv[p] = r
    return np.array(ring, np.int32), np.array(inv, np.int32)


_RING, _INV = _ring_tables()


def kernel(x, w_mat):
    m_per, k = x.shape
    _, n = w_mat.shape
    half = m_per // 2
    piece = half // 2

    ring = jnp.asarray(_RING)
    inv = jnp.asarray(_INV)
    my_pos = lax.axis_index("i")
    my_r = inv[my_pos]
    hs = jnp.arange(N_DEV - 1, dtype=jnp.int32)
    nbrs = jnp.stack([
        ring[(my_r + 1) % N_DEV],
        ring[(my_r - 1) % N_DEV],
    ]).astype(jnp.int32)
    rows_r = (ring[(my_r - hs - 1) % N_DEV] * m_per).astype(jnp.int32)
    rows_l = (ring[(my_r + hs + 1) % N_DEV] * m_per + half).astype(jnp.int32)

    def body(nbr_ref, rows_r_ref, rows_l_ref, x_ref, w_ref, out_ref,
             comm_r, comm_l, send_r, recv_r, send_l, recv_l):
        pos = lax.axis_index("i")
        right = nbr_ref[0]
        left = nbr_ref[1]

        barrier_sem = pltpu.get_barrier_semaphore()
        for nbr in (left, right):
            pl.semaphore_signal(
                barrier_sem, inc=1,
                device_id=(nbr,), device_id_type=pl.DeviceIdType.MESH,
            )
        pl.semaphore_wait(barrier_sem, 2)

        comm_r[0, 0, :, :] = x_ref[:piece, :]
        comm_r[0, 1, :, :] = x_ref[piece:half, :]
        comm_l[0, 0, :, :] = x_ref[half:half + piece, :]
        comm_l[0, 1, :, :] = x_ref[half + piece:, :]

        def desc(h, p, comm, send_sems, recv_sems, dev):
            return pltpu.make_async_remote_copy(
                src_ref=comm.at[h % NSLOT, p],
                dst_ref=comm.at[(h + 1) % NSLOT, p],
                send_sem=send_sems.at[h % NSLOT, p],
                recv_sem=recv_sems.at[(h + 1) % NSLOT, p],
                device_id=(dev,),
                device_id_type=pl.DeviceIdType.MESH,
            )

        def desc_r(h, p):
            return desc(h, p, comm_r, send_r, recv_r, right)

        def desc_l(h, p):
            return desc(h, p, comm_l, send_l, recv_l, left)

        def gemm_round(h):
            slot = (h + 1) % NSLOT
            for p in range(2):
                blk_r = jnp.dot(comm_r[slot, p, :, :], w_ref[:, :],
                                preferred_element_type=jnp.float32)
                out_ref[pl.ds(rows_r_ref[h] + p * piece, piece), :] = blk_r
                blk_l = jnp.dot(comm_l[slot, p, :, :], w_ref[:, :],
                                preferred_element_type=jnp.float32)
                out_ref[pl.ds(rows_l_ref[h] + p * piece, piece), :] = blk_l

        for p in range(2):
            desc_r(0, p).start()
            desc_l(0, p).start()
        own = jnp.dot(x_ref[:, :], w_ref[:, :],
                      preferred_element_type=jnp.float32)
        out_ref[pl.ds(pos * m_per, m_per), :] = own

        for h in range(1, N_DEV - 1):
            for p in range(2):
                desc_r(h - 1, p).wait_recv()
                if h >= NSLOT:
                    desc_r(h - NSLOT, p).wait_send()
                desc_r(h, p).start()
                desc_l(h - 1, p).wait_recv()
                if h >= NSLOT:
                    desc_l(h - NSLOT, p).wait_send()
                desc_l(h, p).start()
            gemm_round(h - 1)

        for p in range(2):
            desc_r(N_DEV - 2, p).wait_recv()
            desc_l(N_DEV - 2, p).wait_recv()
        gemm_round(N_DEV - 2)
        for h in range(N_DEV - 1 - NSLOT, N_DEV - 1):
            for p in range(2):
                desc_r(h, p).wait_send()
                desc_l(h, p).wait_send()

    return pl.pallas_call(
        body,
        out_shape=jax.ShapeDtypeStruct((N_DEV * m_per, n), jnp.float32),
        in_specs=[
            pl.BlockSpec(memory_space=pltpu.SMEM),
            pl.BlockSpec(memory_space=pltpu.SMEM),
            pl.BlockSpec(memory_space=pltpu.SMEM),
            pl.BlockSpec(memory_space=pltpu.VMEM),
            pl.BlockSpec(memory_space=pltpu.VMEM),
        ],
        out_specs=pl.BlockSpec(memory_space=pltpu.VMEM),
        scratch_shapes=[
            pltpu.VMEM((NSLOT, 2, piece, k), jnp.float32),
            pltpu.VMEM((NSLOT, 2, piece, k), jnp.float32),
            pltpu.SemaphoreType.DMA((NSLOT, 2)),
            pltpu.SemaphoreType.DMA((NSLOT, 2)),
            pltpu.SemaphoreType.DMA((NSLOT, 2)),
            pltpu.SemaphoreType.DMA((NSLOT, 2)),
        ],
        compiler_params=pltpu.CompilerParams(collective_id=0),
    )(nbrs, rows_r, rows_l, x, w_mat)
